# baseline (speedup 1.0000x reference)
import functools

import jax
import jax.numpy as jnp
from jax import lax
from jax.experimental import pallas as pl
from jax.experimental.pallas import tpu as pltpu
from jax.experimental.pallas import tpu_sc as plsc

_B, _T, _D = 4096, 200, 64
_L = 16
_NC, _NS = 2, 16
_NW = _NC * _NS
_TT = _T // 8
_GR = _T * _D * _B // _L
_TS = _D * _B // _L


@functools.partial(
    pl.kernel,
    out_type=jax.ShapeDtypeStruct((8, 32, 8, 128), jnp.float32),
    mesh=plsc.VectorSubcoreMesh(core_axis_name="c", subcore_axis_name="s"),
    scratch_types=[
        pltpu.VMEM((_TT, 8, 128), jnp.int32),
        pltpu.VMEM((4, 8, 128), jnp.int32),
        pltpu.VMEM((4, 1024, _L), jnp.float32),
        pltpu.VMEM((8, 1, 8, 128), jnp.float32),
        pltpu.SemaphoreType.DMA,
        pltpu.SemaphoreType.DMA,
        pltpu.SemaphoreType.DMA,
        pltpu.SemaphoreType.DMA,
    ],
    compiler_params=pltpu.CompilerParams(
        needs_layout_passes=False, use_tc_tiling_on_sc=False
    ),
)
def _last_pool_sc(
    a16_hbm, mw_hbm, out_hbm, mbuf, idxbuf, gbuf, obuf, s0, s1, s2, s3
):
    wid = lax.axis_index("s") * _NC + lax.axis_index("c")
    pltpu.sync_copy(mw_hbm.at[:, wid], mbuf)
    zero = jnp.zeros((_L,), jnp.int32)
    lanes = lax.iota(jnp.int32, _L)

    def fill_idx(g, h):
        def tbody(tt, acc):
            row = mbuf.at[tt]
            for tr in range(8):
                acc = acc + row.at[tr][pl.ds(g * _L, _L)]
            return acc

        s = lax.fori_loop(0, _TT, tbody, zero)
        t = jnp.minimum(s, _T - 1) - 1
        t = jnp.where(t < 0, _T - 1, t)
        base = t * _TS + wid * 64 + g

        def kbody(k, _):
            row = idxbuf.at[h].at[k]
            for dl in range(8):
                row[pl.ds(dl * _L, _L)] = base + k * 2048 + dl * 8
            return 0

        lax.fori_loop(0, 8, kbody, 0)

    def fire(h, sem):
        def kbody(k, _):
            pltpu.async_copy(
                a16_hbm.at[idxbuf.at[h].at[k]],
                gbuf.at[h].at[pl.ds(k * 128, 128)],
                sem,
            )
            return 0

        lax.fori_loop(0, 8, kbody, 0)

    def drain(h, sem):
        def kbody(k, _):
            pltpu.make_async_copy(
                a16_hbm.at[idxbuf.at[h].at[k]],
                gbuf.at[h].at[pl.ds(k * 128, 128)],
                sem,
            ).wait()
            return 0

        lax.fori_loop(0, 8, kbody, 0)

    def extract(g, h):
        src = gbuf.at[h]

        def jbody(j, _):
            for dl in range(8):
                val = plsc.load_gather(src, [j * 128 + dl * _L + lanes, lanes])
                obuf.at[j].at[0].at[dl][pl.ds(g * _L, _L)] = val
            return 0

        lax.fori_loop(0, 8, jbody, 0)

    sems = (s0, s1, s2, s3)
    for g in range(3):
        fill_idx(g, g)
        fire(g, sems[g])

    def step(g, _):
        h = g & 3
        for hh in range(4):
            @pl.when(h == hh)
            def _(hh=hh):
                drain(hh, sems[hh])

        extract(g, h)

        @pl.when(g < 5)
        def _():
            hn = (g + 3) & 3
            fill_idx(g + 3, hn)
            for hh in range(4):
                @pl.when(hn == hh)
                def _(hh=hh):
                    fire(hh, sems[hh])

        return 0

    lax.fori_loop(0, 8, step, 0)
    pltpu.sync_copy(obuf, out_hbm.at[:, pl.ds(wid, 1)])


def kernel(x, mask):
    a16 = (
        x.reshape(32, 128, _T, 8, 8)
        .transpose(2, 3, 0, 4, 1)
        .reshape(_GR, _L)
    )
    mask_words = (
        mask.astype(jnp.int32)
        .reshape(32, 128, _TT, 8)
        .transpose(2, 0, 3, 1)
    )
    out_t = _last_pool_sc(a16, mask_words)
    return out_t.transpose(1, 3, 0, 2).reshape(_B, _D)


# speedup vs baseline: 1.0127x; 1.0127x over previous
import functools

import jax
import jax.numpy as jnp
from jax import lax
from jax.experimental import pallas as pl
from jax.experimental.pallas import tpu as pltpu
from jax.experimental.pallas import tpu_sc as plsc

_B, _T, _D = 4096, 200, 64
_L = 16
_NC, _NS = 2, 16
_NW = _NC * _NS
_TT = _T // 8
_GR = _T * _D * _B // _L
_TS = _D * _B // _L


@functools.partial(
    pl.kernel,
    out_type=jax.ShapeDtypeStruct((8, 32, 8, 128), jnp.float32),
    mesh=plsc.VectorSubcoreMesh(core_axis_name="c", subcore_axis_name="s"),
    scratch_types=[
        pltpu.VMEM((_TT, 8, 128), jnp.int32),
        pltpu.VMEM((4, 8, 128), jnp.int32),
        pltpu.VMEM((4, 1024, _L), jnp.float32),
        pltpu.VMEM((8, 1, 8, 128), jnp.float32),
        pltpu.SemaphoreType.DMA,
        pltpu.SemaphoreType.DMA,
        pltpu.SemaphoreType.DMA,
        pltpu.SemaphoreType.DMA,
    ],
    compiler_params=pltpu.CompilerParams(
        needs_layout_passes=False, use_tc_tiling_on_sc=False
    ),
)
def _last_pool_sc(
    a16_hbm, mw_hbm, out_hbm, mbuf, idxbuf, gbuf, obuf, s0, s1, s2, s3
):
    wid = lax.axis_index("s") * _NC + lax.axis_index("c")
    pltpu.sync_copy(mw_hbm.at[:, wid], mbuf)
    zero = jnp.zeros((_L,), jnp.int32)
    lanes = lax.iota(jnp.int32, _L)

    def fill_idx(g, h):
        def tbody(tt, acc):
            row = mbuf.at[tt]
            for tr in range(8):
                acc = acc + row.at[tr][pl.ds(g * _L, _L)]
            return acc

        s = lax.fori_loop(0, _TT, tbody, zero)
        t = jnp.minimum(s, _T - 1) - 1
        t = jnp.where(t < 0, _T - 1, t)
        base = t * _TS + wid * 64 + g

        def kbody(k, _):
            row = idxbuf.at[h].at[k]
            for dl in range(8):
                row[pl.ds(dl * _L, _L)] = base + k * 2048 + dl * 8
            return 0

        lax.fori_loop(0, 8, kbody, 0)

    def fire(h, sem):
        def kbody(k, _):
            pltpu.async_copy(
                a16_hbm.at[idxbuf.at[h].at[k]],
                gbuf.at[h].at[pl.ds(k * 128, 128)],
                sem,
            )
            return 0

        lax.fori_loop(0, 8, kbody, 0)

    def drain(h, sem):
        def kbody(k, _):
            pltpu.make_async_copy(
                a16_hbm.at[idxbuf.at[h].at[k]],
                gbuf.at[h].at[pl.ds(k * 128, 128)],
                sem,
            ).wait()
            return 0

        lax.fori_loop(0, 8, kbody, 0)

    def extract(g, h):
        src = gbuf.at[h]

        def jbody(j, _):
            for dl in range(8):
                val = plsc.load_gather(src, [j * 128 + dl * _L + lanes, lanes])
                obuf.at[j].at[0].at[dl][pl.ds(g * _L, _L)] = val
            return 0

        lax.fori_loop(0, 8, jbody, 0)

    sems = (s0, s1, s2, s3)
    for g in range(3):
        fill_idx(g, g)
        fire(g, sems[g])

    def step(g, _):
        h = g & 3

        @pl.when(g < 5)
        def _():
            hn = (g + 3) & 3
            fill_idx(g + 3, hn)
            for hh in range(4):
                @pl.when(hn == hh)
                def _(hh=hh):
                    fire(hh, sems[hh])

        for hh in range(4):
            @pl.when(h == hh)
            def _(hh=hh):
                drain(hh, sems[hh])

        extract(g, h)
        return 0

    lax.fori_loop(0, 8, step, 0)
    pltpu.sync_copy(obuf, out_hbm.at[:, pl.ds(wid, 1)])


def kernel(x, mask):
    a16 = (
        x.reshape(32, 128, _T, 8, 8)
        .transpose(2, 3, 0, 4, 1)
        .reshape(_GR, _L)
    )
    mask_words = (
        mask.astype(jnp.int32)
        .reshape(32, 128, _TT, 8)
        .transpose(2, 0, 3, 1)
    )
    out_t = _last_pool_sc(a16, mask_words)
    return out_t.transpose(1, 3, 0, 2).reshape(_B, _D)
